# Initial kernel scaffold; baseline (speedup 1.0000x reference)
import jax
import jax.numpy as jnp
from jax.experimental import pallas as pl


def kernel(x, edge_index, weight, w_ih, w_hh, b_ih, b_hh):
    raise NotImplementedError("write your pallas kernel here")


# speedup vs baseline: 6.2409x; 6.2409x over previous
import functools

import jax
import jax.numpy as jnp
from jax import lax
from jax.experimental import pallas as pl
from jax.experimental.pallas import tpu as pltpu
from jax.experimental.pallas import tpu_sc as plsc

N = 10000
D = 128
E = 320000
NUM_LAYERS = 3

NC = 2
NS = 16
NW = NC * NS
EPW = E // NW
CHUNK = 80
NCH = EPW // CHUNK
NP = 10240
RPT = NP // NS


def _sc_segment_sum(m, src3, dst3, zeros):
    mesh = plsc.VectorSubcoreMesh(core_axis_name="c", subcore_axis_name="s")

    @functools.partial(
        pl.kernel,
        out_type=jax.ShapeDtypeStruct((NC * NP, D), jnp.float32),
        mesh=mesh,
        scratch_types=[
            pltpu.VMEM((NCH, CHUNK), jnp.int32),
            pltpu.VMEM((NCH, CHUNK), jnp.int32),
            pltpu.VMEM((CHUNK, D), jnp.float32),
            pltpu.VMEM_SHARED((NP, D), jnp.float32),
            pltpu.SemaphoreType.DMA,
        ],
    )
    def seg(m_hbm, src_hbm, dst_hbm, z_hbm, out_hbm, src_v, dst_v, rows_v,
            acc_sh, sem):
        cid = lax.axis_index("c")
        sid = lax.axis_index("s")
        wid = sid * NC + cid
        pltpu.sync_copy(src_hbm.at[wid], src_v)
        pltpu.sync_copy(dst_hbm.at[wid], dst_v)
        row0 = sid * RPT
        pltpu.sync_copy(z_hbm.at[pl.ds(row0, RPT)], acc_sh.at[pl.ds(row0, RPT)])
        plsc.subcore_barrier()

        def body(j, carry):
            pltpu.async_copy(m_hbm.at[src_v.at[j]], rows_v, sem).wait()
            pltpu.sync_copy(rows_v, acc_sh.at[dst_v.at[j]], add=True)
            return carry

        lax.fori_loop(0, NCH, body, 0)
        plsc.subcore_barrier()
        pltpu.sync_copy(acc_sh.at[pl.ds(row0, RPT)],
                        out_hbm.at[pl.ds(cid * NP + row0, RPT)])

    return seg(m, src3, dst3, zeros)


BLK = 1000


def _gru_proj_body(p0, p1, h, wih, whh, bih, bhh, wn, h_out, m_out):
    agg = p0[...] + p1[...]
    gi = jnp.dot(agg, wih[...], preferred_element_type=jnp.float32) + bih[...]
    gh = jnp.dot(h[...], whh[...], preferred_element_type=jnp.float32) + bhh[...]
    r = jax.nn.sigmoid(gi[:, :D] + gh[:, :D])
    z = jax.nn.sigmoid(gi[:, D:2 * D] + gh[:, D:2 * D])
    n = jnp.tanh(gi[:, 2 * D:] + r * gh[:, 2 * D:])
    hn = (1.0 - z) * n + z * h[...]
    h_out[...] = hn
    m_out[...] = jnp.dot(hn, wn[...], preferred_element_type=jnp.float32)


def _tc_gru_proj(p0, p1, h, wihT, whhT, bih, bhh, wnext):
    row = pl.BlockSpec((BLK, D), lambda i: (i, 0))
    full = lambda shape: pl.BlockSpec(shape, lambda i: (0,) * len(shape))
    return pl.pallas_call(
        _gru_proj_body,
        grid=(N // BLK,),
        in_specs=[row, row, row,
                  full((D, 3 * D)), full((D, 3 * D)),
                  full((1, 3 * D)), full((1, 3 * D)),
                  full((D, D))],
        out_specs=(row, row),
        out_shape=(jax.ShapeDtypeStruct((N, D), jnp.float32),
                   jax.ShapeDtypeStruct((N, D), jnp.float32)),
    )(p0, p1, h, wihT, whhT, bih, bhh, wnext)


def _proj_body(h, w, m_out):
    m_out[...] = jnp.dot(h[...], w[...], preferred_element_type=jnp.float32)


def _tc_proj(h, w):
    row = pl.BlockSpec((BLK, D), lambda i: (i, 0))
    return pl.pallas_call(
        _proj_body,
        grid=(N // BLK,),
        in_specs=[row, pl.BlockSpec((D, D), lambda i: (0, 0))],
        out_specs=row,
        out_shape=jax.ShapeDtypeStruct((N, D), jnp.float32),
    )(h, w)


def kernel(x, edge_index, weight, w_ih, w_hh, b_ih, b_hh):
    src3 = edge_index[0].reshape(NW, NCH, CHUNK)
    dst3 = edge_index[1].reshape(NW, NCH, CHUNK)
    zeros = jnp.zeros((NP, D), jnp.float32)
    wihT = jnp.transpose(w_ih, (0, 2, 1))
    whhT = jnp.transpose(w_hh, (0, 2, 1))
    bih2 = b_ih.reshape(NUM_LAYERS, 1, 3 * D)
    bhh2 = b_hh.reshape(NUM_LAYERS, 1, 3 * D)

    h = x
    m = _tc_proj(h, weight[0])
    for i in range(NUM_LAYERS):
        parts = _sc_segment_sum(m, src3, dst3, zeros)
        wnext = weight[i + 1] if i + 1 < NUM_LAYERS else weight[0]
        h, m = _tc_gru_proj(parts[:N], parts[NP:NP + N], h, wihT[i], whhT[i],
                            bih2[i], bhh2[i], wnext)
    return h
